# Initial kernel scaffold; baseline (speedup 1.0000x reference)
import jax
import jax.numpy as jnp
from jax.experimental import pallas as pl


def kernel(x, edge_index, W1, b1, W2, b2):
    raise NotImplementedError("write your pallas kernel here")


# speedup vs baseline: 15.0392x; 15.0392x over previous
import functools

import jax
import jax.numpy as jnp
from jax import lax
from jax.experimental import pallas as pl
from jax.experimental.pallas import tpu as pltpu
from jax.experimental.pallas import tpu_sc as plsc

N_NODES = 10000
N_EDGES = 320000
D_IN = 128
D_HID = 128
N_CLASSES = 40

NPAD = 10240
CW = 128
CHUNK = 128
NW = 32
NCHUNK = N_EDGES // CHUNK
CPT = NCHUNK // NW
REM = NCHUNK - CPT * NW
NSUB = 16
RPT = NPAD // NSUB


def _zero_rows(buf, nrows, width):
    def outer(r, _):
        def inner(j, _):
            buf[r, pl.ds(j * 16, 16)] = jnp.zeros((16,), jnp.float32)
            return 0
        lax.fori_loop(0, width // 16, inner, 0)
        return 0
    lax.fori_loop(0, nrows, outer, 0)


def _deg_kernel(dst1d):
    mesh = plsc.VectorSubcoreMesh(core_axis_name="c", subcore_axis_name="s")

    @functools.partial(
        pl.kernel,
        out_type=jax.ShapeDtypeStruct((2, NPAD), jnp.float32),
        mesh=mesh,
        scratch_types=[
            pltpu.VMEM((CHUNK,), jnp.int32),
            pltpu.VMEM((CHUNK,), jnp.float32),
            pltpu.VMEM((RPT,), jnp.float32),
            pltpu.VMEM_SHARED((NPAD,), jnp.float32),
        ],
    )
    def deg(dst_hbm, out_hbm, di, ones_v, row_v, acc):
        c = lax.axis_index("c")
        s = lax.axis_index("s")
        wid = c * NSUB + s

        def fill(i, _):
            ones_v[pl.ds(i * 16, 16)] = jnp.ones((16,), jnp.float32)
            return 0
        lax.fori_loop(0, CHUNK // 16, fill, 0)

        def fill0(i, _):
            row_v[pl.ds(i * 16, 16)] = jnp.zeros((16,), jnp.float32)
            return 0
        lax.fori_loop(0, RPT // 16, fill0, 0)
        pltpu.sync_copy(row_v, acc.at[pl.ds(s * RPT, RPT)])
        plsc.subcore_barrier()

        def one(t):
            pltpu.sync_copy(dst_hbm.at[pl.ds(t * CHUNK, CHUNK)], di)
            pltpu.sync_copy(ones_v, acc.at[di], add=True)

        def body(t, _):
            one(t)
            return 0
        lax.fori_loop(wid * CPT, (wid + 1) * CPT, body, 0)

        @pl.when(wid < REM)
        def _():
            one(NW * CPT + wid)

        plsc.subcore_barrier()
        pltpu.sync_copy(acc.at[pl.ds(s * RPT, RPT)], row_v)
        pltpu.sync_copy(row_v, out_hbm.at[c, pl.ds(s * RPT, RPT)])

    return deg(dst1d)


def _edge_scatter(g, src1d, dst1d, width):
    mesh = plsc.VectorSubcoreMesh(core_axis_name="c", subcore_axis_name="s")

    @functools.partial(
        pl.kernel,
        out_type=jax.ShapeDtypeStruct((2, NPAD, width), jnp.float32),
        mesh=mesh,
        scratch_types=[
            pltpu.VMEM((CHUNK,), jnp.int32),
            pltpu.VMEM((CHUNK,), jnp.int32),
            pltpu.VMEM((CHUNK, width), jnp.float32),
            pltpu.SemaphoreType.DMA,
            pltpu.VMEM_SHARED((NPAD, width), jnp.float32),
        ],
    )
    def scat(g_hbm, src_hbm, dst_hbm, out_hbm,
             si, di, rows0, sem0, acc):
        c = lax.axis_index("c")
        s = lax.axis_index("s")
        wid = c * NSUB + s

        _zero_rows(rows0, CHUNK, width)
        def z(k, _):
            pltpu.sync_copy(rows0, acc.at[pl.ds(s * RPT + k * CHUNK, CHUNK)])
            return 0
        lax.fori_loop(0, RPT // CHUNK, z, 0)
        plsc.subcore_barrier()

        def one(t):
            base = t * CHUNK
            pltpu.sync_copy(src_hbm.at[pl.ds(base, CHUNK)], si)
            pltpu.sync_copy(dst_hbm.at[pl.ds(base, CHUNK)], di)
            pltpu.async_copy(g_hbm.at[si], rows0, sem0).wait()
            pltpu.sync_copy(rows0, acc.at[di], add=True)

        def body(t, _):
            one(t)
            return 0
        lax.fori_loop(wid * CPT, (wid + 1) * CPT, body, 0)

        @pl.when(wid < REM)
        def _():
            one(NW * CPT + wid)

        plsc.subcore_barrier()

        def rd(k, _):
            r0 = s * RPT + k * CHUNK
            pltpu.sync_copy(acc.at[pl.ds(r0, CHUNK)], rows0)
            pltpu.sync_copy(rows0, out_hbm.at[c, pl.ds(r0, CHUNK)])
            return 0
        lax.fori_loop(0, RPT // CHUNK, rd, 0)

    return scat(g, src1d, dst1d)


_BM = 256


def _tc_scale_matmul(xp, w, dinv_col):
    k = xp.shape[1]
    n = w.shape[1]

    def body(x_ref, w_ref, d_ref, o_ref):
        h = jnp.dot(x_ref[...], w_ref[...], preferred_element_type=jnp.float32)
        o_ref[...] = h * d_ref[...]

    return pl.pallas_call(
        body,
        grid=(NPAD // _BM,),
        in_specs=[
            pl.BlockSpec((_BM, k), lambda i: (i, 0)),
            pl.BlockSpec((k, n), lambda i: (0, 0)),
            pl.BlockSpec((_BM, 1), lambda i: (i, 0)),
        ],
        out_specs=pl.BlockSpec((_BM, n), lambda i: (i, 0)),
        out_shape=jax.ShapeDtypeStruct((NPAD, n), jnp.float32),
    )(xp, w, dinv_col)


def _tc_mid(parts, g1, dinv_col, b1row, w2):
    n = w2.shape[1]

    def body(p_ref, g_ref, d_ref, b_ref, w_ref, o_ref):
        d = d_ref[...]
        z = (p_ref[0] + p_ref[1] + g_ref[...]) * d + b_ref[...]
        z = jnp.maximum(z, 0.0)
        o_ref[...] = jnp.dot(z, w_ref[...], preferred_element_type=jnp.float32) * d

    return pl.pallas_call(
        body,
        grid=(NPAD // _BM,),
        in_specs=[
            pl.BlockSpec((2, _BM, D_HID), lambda i: (0, i, 0)),
            pl.BlockSpec((_BM, D_HID), lambda i: (i, 0)),
            pl.BlockSpec((_BM, 1), lambda i: (i, 0)),
            pl.BlockSpec((1, D_HID), lambda i: (0, 0)),
            pl.BlockSpec((D_HID, n), lambda i: (0, 0)),
        ],
        out_specs=pl.BlockSpec((_BM, n), lambda i: (i, 0)),
        out_shape=jax.ShapeDtypeStruct((NPAD, n), jnp.float32),
    )(parts, g1, dinv_col, b1row, w2)


def _tc_final(parts, g2, dinv_col, b2row):
    n = g2.shape[1]

    def body(p_ref, g_ref, d_ref, b_ref, o_ref):
        o_ref[...] = (p_ref[0] + p_ref[1] + g_ref[...]) * d_ref[...] + b_ref[...]

    return pl.pallas_call(
        body,
        grid=(NPAD // _BM,),
        in_specs=[
            pl.BlockSpec((2, _BM, n), lambda i: (0, i, 0)),
            pl.BlockSpec((_BM, n), lambda i: (i, 0)),
            pl.BlockSpec((_BM, 1), lambda i: (i, 0)),
            pl.BlockSpec((1, n), lambda i: (0, 0)),
        ],
        out_specs=pl.BlockSpec((_BM, n), lambda i: (i, 0)),
        out_shape=jax.ShapeDtypeStruct((NPAD, n), jnp.float32),
    )(parts, g2, dinv_col, b2row)


@jax.jit
def kernel(x, edge_index, W1, b1, W2, b2):
    ei = edge_index.astype(jnp.int32)
    src1d = ei[0]
    dst1d = ei[1]
    xp = jnp.pad(x, ((0, NPAD - N_NODES), (0, 0)))

    deg = _deg_kernel(dst1d)
    dinv_col = lax.rsqrt(deg[0] + deg[1] + 1.0)[:, None]

    g1 = _tc_scale_matmul(xp, W1, dinv_col)
    parts1 = _edge_scatter(g1, src1d, dst1d, D_HID)

    w2p = jnp.pad(W2, ((0, 0), (0, CW - N_CLASSES)))
    b2p = jnp.pad(b2, (0, CW - N_CLASSES))
    g2 = _tc_mid(parts1, g1, dinv_col, b1[None, :], w2p)
    parts2 = _edge_scatter(g2, src1d, dst1d, CW)

    outp = _tc_final(parts2, g2, dinv_col, b2p[None, :])
    return outp[:N_NODES, :N_CLASSES]


# speedup vs baseline: 25.9188x; 1.7234x over previous
import functools

import jax
import jax.numpy as jnp
from jax import lax
from jax.experimental import pallas as pl
from jax.experimental.pallas import tpu as pltpu
from jax.experimental.pallas import tpu_sc as plsc

N_NODES = 10000
N_EDGES = 320000
D_IN = 128
D_HID = 128
N_CLASSES = 40

NPAD = 10240
CW = 128
CHUNK = 128
NW = 32
NCHUNK = N_EDGES // CHUNK
CPT = NCHUNK // NW
REM = NCHUNK - CPT * NW
NSUB = 16
RPT = NPAD // NSUB


def _zero_rows(buf, nrows, width):
    def outer(r, _):
        def inner(j, _):
            buf[r, pl.ds(j * 16, 16)] = jnp.zeros((16,), jnp.float32)
            return 0
        lax.fori_loop(0, width // 16, inner, 0)
        return 0
    lax.fori_loop(0, nrows, outer, 0)


def _deg_kernel(dst1d):
    mesh = plsc.VectorSubcoreMesh(core_axis_name="c", subcore_axis_name="s")

    @functools.partial(
        pl.kernel,
        out_type=jax.ShapeDtypeStruct((2, NPAD), jnp.float32),
        mesh=mesh,
        scratch_types=[
            pltpu.VMEM((CHUNK,), jnp.int32),
            pltpu.VMEM((CHUNK,), jnp.float32),
            pltpu.VMEM((RPT,), jnp.float32),
            pltpu.VMEM_SHARED((NPAD,), jnp.float32),
        ],
    )
    def deg(dst_hbm, out_hbm, di, ones_v, row_v, acc):
        c = lax.axis_index("c")
        s = lax.axis_index("s")
        wid = c * NSUB + s

        def fill(i, _):
            ones_v[pl.ds(i * 16, 16)] = jnp.ones((16,), jnp.float32)
            return 0
        lax.fori_loop(0, CHUNK // 16, fill, 0)

        def fill0(i, _):
            row_v[pl.ds(i * 16, 16)] = jnp.zeros((16,), jnp.float32)
            return 0
        lax.fori_loop(0, RPT // 16, fill0, 0)
        pltpu.sync_copy(row_v, acc.at[pl.ds(s * RPT, RPT)])
        plsc.subcore_barrier()

        def one(t):
            pltpu.sync_copy(dst_hbm.at[pl.ds(t * CHUNK, CHUNK)], di)
            pltpu.sync_copy(ones_v, acc.at[di], add=True)

        def body(t, _):
            one(t)
            return 0
        lax.fori_loop(wid * CPT, (wid + 1) * CPT, body, 0)

        @pl.when(wid < REM)
        def _():
            one(NW * CPT + wid)

        plsc.subcore_barrier()
        pltpu.sync_copy(acc.at[pl.ds(s * RPT, RPT)], row_v)
        pltpu.sync_copy(row_v, out_hbm.at[c, pl.ds(s * RPT, RPT)])

    return deg(dst1d)


def _edge_scatter(g, src1d, dst1d, width):
    mesh = plsc.VectorSubcoreMesh(core_axis_name="c", subcore_axis_name="s")

    @functools.partial(
        pl.kernel,
        out_type=jax.ShapeDtypeStruct((2, NPAD, width), jnp.float32),
        mesh=mesh,
        scratch_types=[
            pltpu.VMEM((CPT * CHUNK,), jnp.int32),
            pltpu.VMEM((CHUNK,), jnp.int32),
            pltpu.VMEM((CHUNK,), jnp.int32),
            pltpu.VMEM((CHUNK,), jnp.int32),
            pltpu.VMEM((CHUNK, width), jnp.float32),
            pltpu.VMEM((CHUNK, width), jnp.float32),
            pltpu.SemaphoreType.DMA,
            pltpu.SemaphoreType.DMA,
            pltpu.SemaphoreType.DMA,
            pltpu.SemaphoreType.DMA,
            pltpu.VMEM_SHARED((NPAD, width), jnp.float32),
        ],
    )
    def scat(g_hbm, src_hbm, dst_hbm, out_hbm,
             sb, sx, di0, di1, rows0, rows1, sg0, sg1, sd0, sd1, acc):
        c = lax.axis_index("c")
        s = lax.axis_index("s")
        wid = c * NSUB + s
        e0 = wid * CPT * CHUNK

        _zero_rows(rows0, CHUNK, width)
        def z(k, _):
            pltpu.sync_copy(rows0, acc.at[pl.ds(s * RPT + k * CHUNK, CHUNK)])
            return 0
        lax.fori_loop(0, RPT // CHUNK, z, 0)
        plsc.subcore_barrier()

        pltpu.sync_copy(src_hbm.at[pl.ds(e0, CPT * CHUNK)], sb)

        def src_at(t):
            return sb.at[pl.ds(t * CHUNK, CHUNK)]

        def start_gather(t, rows, sem):
            return pltpu.async_copy(g_hbm.at[src_at(t)], rows, sem)

        def start_dst(t, di, sem):
            return pltpu.async_copy(
                dst_hbm.at[pl.ds(e0 + t * CHUNK, CHUNK)], di, sem)

        start_dst(0, di0, sd0)
        start_dst(1, di1, sd1)
        start_gather(0, rows0, sg0)
        start_gather(1, rows1, sg1)

        def slot(i, t, rows, di, sg, sd, nxt_valid):
            pltpu.make_async_copy(g_hbm.at[src_at(t)], rows, sg).wait()
            pltpu.make_async_copy(dst_hbm.at[pl.ds(e0, CHUNK)], di, sd).wait()
            pltpu.sync_copy(rows, acc.at[di], add=True)

            @pl.when(nxt_valid)
            def _():
                start_dst(t + 2, di, sd)
                start_gather(t + 2, rows, sg)

        def body(i, _):
            t0 = 2 * i
            slot(i, t0, rows0, di0, sg0, sd0, t0 + 2 < CPT)
            slot(i, t0 + 1, rows1, di1, sg1, sd1, t0 + 3 < CPT)
            return 0
        lax.fori_loop(0, CPT // 2, body, 0)

        @pl.when(wid < REM)
        def _():
            base = (NW * CPT + wid) * CHUNK
            pltpu.sync_copy(src_hbm.at[pl.ds(base, CHUNK)], sx)
            pltpu.sync_copy(dst_hbm.at[pl.ds(base, CHUNK)], di0)
            pltpu.async_copy(g_hbm.at[sx], rows0, sg0).wait()
            pltpu.sync_copy(rows0, acc.at[di0], add=True)

        plsc.subcore_barrier()

        def rd(k, _):
            r0 = s * RPT + k * CHUNK
            pltpu.sync_copy(acc.at[pl.ds(r0, CHUNK)], rows0)
            pltpu.sync_copy(rows0, out_hbm.at[c, pl.ds(r0, CHUNK)])
            return 0
        lax.fori_loop(0, RPT // CHUNK, rd, 0)

    return scat(g, src1d, dst1d)


_BM = 256


def _tc_scale_matmul(xp, w, dinv_col):
    k = xp.shape[1]
    n = w.shape[1]

    def body(x_ref, w_ref, d_ref, o_ref):
        h = jnp.dot(x_ref[...], w_ref[...], preferred_element_type=jnp.float32)
        o_ref[...] = h * d_ref[...]

    return pl.pallas_call(
        body,
        grid=(NPAD // _BM,),
        in_specs=[
            pl.BlockSpec((_BM, k), lambda i: (i, 0)),
            pl.BlockSpec((k, n), lambda i: (0, 0)),
            pl.BlockSpec((_BM, 1), lambda i: (i, 0)),
        ],
        out_specs=pl.BlockSpec((_BM, n), lambda i: (i, 0)),
        out_shape=jax.ShapeDtypeStruct((NPAD, n), jnp.float32),
    )(xp, w, dinv_col)


def _tc_mid(parts, g1, dinv_col, b1row, w2):
    n = w2.shape[1]

    def body(p_ref, g_ref, d_ref, b_ref, w_ref, o_ref):
        d = d_ref[...]
        z = (p_ref[0] + p_ref[1] + g_ref[...]) * d + b_ref[...]
        z = jnp.maximum(z, 0.0)
        o_ref[...] = jnp.dot(z, w_ref[...], preferred_element_type=jnp.float32) * d

    return pl.pallas_call(
        body,
        grid=(NPAD // _BM,),
        in_specs=[
            pl.BlockSpec((2, _BM, D_HID), lambda i: (0, i, 0)),
            pl.BlockSpec((_BM, D_HID), lambda i: (i, 0)),
            pl.BlockSpec((_BM, 1), lambda i: (i, 0)),
            pl.BlockSpec((1, D_HID), lambda i: (0, 0)),
            pl.BlockSpec((D_HID, n), lambda i: (0, 0)),
        ],
        out_specs=pl.BlockSpec((_BM, n), lambda i: (i, 0)),
        out_shape=jax.ShapeDtypeStruct((NPAD, n), jnp.float32),
    )(parts, g1, dinv_col, b1row, w2)


def _tc_final(parts, g2, dinv_col, b2row):
    n = g2.shape[1]

    def body(p_ref, g_ref, d_ref, b_ref, o_ref):
        o_ref[...] = (p_ref[0] + p_ref[1] + g_ref[...]) * d_ref[...] + b_ref[...]

    return pl.pallas_call(
        body,
        grid=(NPAD // _BM,),
        in_specs=[
            pl.BlockSpec((2, _BM, n), lambda i: (0, i, 0)),
            pl.BlockSpec((_BM, n), lambda i: (i, 0)),
            pl.BlockSpec((_BM, 1), lambda i: (i, 0)),
            pl.BlockSpec((1, n), lambda i: (0, 0)),
        ],
        out_specs=pl.BlockSpec((_BM, n), lambda i: (i, 0)),
        out_shape=jax.ShapeDtypeStruct((NPAD, n), jnp.float32),
    )(parts, g2, dinv_col, b2row)


@jax.jit
def kernel(x, edge_index, W1, b1, W2, b2):
    ei = edge_index.astype(jnp.int32)
    src1d = ei[0]
    dst1d = ei[1]
    xp = jnp.pad(x, ((0, NPAD - N_NODES), (0, 0)))

    deg = _deg_kernel(dst1d)
    dinv_col = lax.rsqrt(deg[0] + deg[1] + 1.0)[:, None]

    g1 = _tc_scale_matmul(xp, W1, dinv_col)
    parts1 = _edge_scatter(g1, src1d, dst1d, D_HID)

    w2p = jnp.pad(W2, ((0, 0), (0, CW - N_CLASSES)))
    b2p = jnp.pad(b2, (0, CW - N_CLASSES))
    g2 = _tc_mid(parts1, g1, dinv_col, b1[None, :], w2p)
    parts2 = _edge_scatter(g2, src1d, dst1d, CW)

    outp = _tc_final(parts2, g2, dinv_col, b2p[None, :])
    return outp[:N_NODES, :N_CLASSES]


# speedup vs baseline: 28.8259x; 1.1122x over previous
import functools

import jax
import jax.numpy as jnp
from jax import lax
from jax.experimental import pallas as pl
from jax.experimental.pallas import tpu as pltpu
from jax.experimental.pallas import tpu_sc as plsc

N_NODES = 10000
N_EDGES = 320000
D_IN = 128
D_HID = 128
N_CLASSES = 40

NPAD = 10240
CW = 128
CHUNK = 128
NW = 32
NCHUNK = N_EDGES // CHUNK
CPT = NCHUNK // NW
REM = NCHUNK - CPT * NW
NSUB = 16
RPT = NPAD // NSUB


def _zero_rows(buf, nrows, width):
    def outer(r, _):
        def inner(j, _):
            buf[r, pl.ds(j * 16, 16)] = jnp.zeros((16,), jnp.float32)
            return 0
        lax.fori_loop(0, width // 16, inner, 0)
        return 0
    lax.fori_loop(0, nrows, outer, 0)


def _deg_kernel(dst1d):
    mesh = plsc.VectorSubcoreMesh(core_axis_name="c", subcore_axis_name="s")

    @functools.partial(
        pl.kernel,
        out_type=jax.ShapeDtypeStruct((2, NPAD), jnp.float32),
        mesh=mesh,
        scratch_types=[
            pltpu.VMEM((CPT, CHUNK), jnp.int32),
            pltpu.VMEM((CHUNK,), jnp.int32),
            pltpu.VMEM((CHUNK,), jnp.float32),
            pltpu.VMEM((RPT,), jnp.float32),
            pltpu.SemaphoreType.DMA,
            pltpu.SemaphoreType.DMA,
            pltpu.VMEM_SHARED((NPAD,), jnp.float32),
        ],
    )
    def deg(dst_hbm, out_hbm, di2, dx, ones_v, row_v, sem, sem2, acc):
        c = lax.axis_index("c")
        s = lax.axis_index("s")
        wid = c * NSUB + s
        e0 = wid * CPT * CHUNK

        def ld(t, _):
            pltpu.async_copy(
                dst_hbm.at[pl.ds(e0 + t * CHUNK, CHUNK)], di2.at[t], sem)
            return 0
        lax.fori_loop(0, CPT, ld, 0)

        def fill(i, _):
            ones_v[pl.ds(i * 16, 16)] = jnp.ones((16,), jnp.float32)
            return 0
        lax.fori_loop(0, CHUNK // 16, fill, 0)

        def fill0(i, _):
            row_v[pl.ds(i * 16, 16)] = jnp.zeros((16,), jnp.float32)
            return 0
        lax.fori_loop(0, RPT // 16, fill0, 0)
        pltpu.sync_copy(row_v, acc.at[pl.ds(s * RPT, RPT)])

        def dr(t, _):
            pltpu.make_async_copy(
                dst_hbm.at[pl.ds(e0, CHUNK)], di2.at[t], sem).wait()
            return 0
        lax.fori_loop(0, CPT, dr, 0)
        plsc.subcore_barrier()

        def sc(t, _):
            pltpu.async_copy(ones_v, acc.at[di2.at[t]], sem2, add=True)
            return 0
        lax.fori_loop(0, CPT, sc, 0)

        @pl.when(wid < REM)
        def _():
            base = (NW * CPT + wid) * CHUNK
            pltpu.sync_copy(dst_hbm.at[pl.ds(base, CHUNK)], dx)
            pltpu.sync_copy(ones_v, acc.at[dx], add=True)

        def dr2(t, _):
            pltpu.make_async_copy(ones_v, acc.at[di2.at[t]], sem2).wait()
            return 0
        lax.fori_loop(0, CPT, dr2, 0)
        plsc.subcore_barrier()
        pltpu.sync_copy(acc.at[pl.ds(s * RPT, RPT)], row_v)
        pltpu.sync_copy(row_v, out_hbm.at[c, pl.ds(s * RPT, RPT)])

    return deg(dst1d)


def _edge_scatter(g, src1d, dst1d, width):
    mesh = plsc.VectorSubcoreMesh(core_axis_name="c", subcore_axis_name="s")

    @functools.partial(
        pl.kernel,
        out_type=jax.ShapeDtypeStruct((2, NPAD, width), jnp.float32),
        mesh=mesh,
        scratch_types=[
            pltpu.VMEM((CPT * CHUNK,), jnp.int32),
            pltpu.VMEM((CHUNK,), jnp.int32),
            pltpu.VMEM((CHUNK,), jnp.int32),
            pltpu.VMEM((CHUNK,), jnp.int32),
            pltpu.VMEM((CHUNK, width), jnp.float32),
            pltpu.VMEM((CHUNK, width), jnp.float32),
            pltpu.SemaphoreType.DMA,
            pltpu.SemaphoreType.DMA,
            pltpu.SemaphoreType.DMA,
            pltpu.SemaphoreType.DMA,
            pltpu.VMEM_SHARED((NPAD, width), jnp.float32),
        ],
    )
    def scat(g_hbm, src_hbm, dst_hbm, out_hbm,
             sb, sx, di0, di1, rows0, rows1, sg0, sg1, sd0, sd1, acc):
        c = lax.axis_index("c")
        s = lax.axis_index("s")
        wid = c * NSUB + s
        e0 = wid * CPT * CHUNK

        _zero_rows(rows0, CHUNK, width)
        def z(k, _):
            pltpu.sync_copy(rows0, acc.at[pl.ds(s * RPT + k * CHUNK, CHUNK)])
            return 0
        lax.fori_loop(0, RPT // CHUNK, z, 0)
        plsc.subcore_barrier()

        pltpu.sync_copy(src_hbm.at[pl.ds(e0, CPT * CHUNK)], sb)

        def src_at(t):
            return sb.at[pl.ds(t * CHUNK, CHUNK)]

        def start_gather(t, rows, sem):
            return pltpu.async_copy(g_hbm.at[src_at(t)], rows, sem)

        def start_dst(t, di, sem):
            return pltpu.async_copy(
                dst_hbm.at[pl.ds(e0 + t * CHUNK, CHUNK)], di, sem)

        start_dst(0, di0, sd0)
        start_dst(1, di1, sd1)
        start_gather(0, rows0, sg0)
        start_gather(1, rows1, sg1)

        def slot(i, t, rows, di, sg, sd, nxt_valid):
            pltpu.make_async_copy(g_hbm.at[src_at(t)], rows, sg).wait()
            pltpu.make_async_copy(dst_hbm.at[pl.ds(e0, CHUNK)], di, sd).wait()
            pltpu.sync_copy(rows, acc.at[di], add=True)

            @pl.when(nxt_valid)
            def _():
                start_dst(t + 2, di, sd)
                start_gather(t + 2, rows, sg)

        def body(i, _):
            t0 = 2 * i
            slot(i, t0, rows0, di0, sg0, sd0, t0 + 2 < CPT)
            slot(i, t0 + 1, rows1, di1, sg1, sd1, t0 + 3 < CPT)
            return 0
        lax.fori_loop(0, CPT // 2, body, 0)

        @pl.when(wid < REM)
        def _():
            base = (NW * CPT + wid) * CHUNK
            pltpu.sync_copy(src_hbm.at[pl.ds(base, CHUNK)], sx)
            pltpu.sync_copy(dst_hbm.at[pl.ds(base, CHUNK)], di0)
            pltpu.async_copy(g_hbm.at[sx], rows0, sg0).wait()
            pltpu.sync_copy(rows0, acc.at[di0], add=True)

        plsc.subcore_barrier()

        def rd(k, _):
            r0 = s * RPT + k * CHUNK
            pltpu.sync_copy(acc.at[pl.ds(r0, CHUNK)], rows0)
            pltpu.sync_copy(rows0, out_hbm.at[c, pl.ds(r0, CHUNK)])
            return 0
        lax.fori_loop(0, RPT // CHUNK, rd, 0)

    return scat(g, src1d, dst1d)


_BM = 256


def _tc_scale_matmul(xp, w, dinv_col):
    k = xp.shape[1]
    n = w.shape[1]

    def body(x_ref, w_ref, d_ref, o_ref):
        h = jnp.dot(x_ref[...], w_ref[...], preferred_element_type=jnp.float32)
        o_ref[...] = h * d_ref[...]

    return pl.pallas_call(
        body,
        grid=(NPAD // _BM,),
        in_specs=[
            pl.BlockSpec((_BM, k), lambda i: (i, 0)),
            pl.BlockSpec((k, n), lambda i: (0, 0)),
            pl.BlockSpec((_BM, 1), lambda i: (i, 0)),
        ],
        out_specs=pl.BlockSpec((_BM, n), lambda i: (i, 0)),
        out_shape=jax.ShapeDtypeStruct((NPAD, n), jnp.float32),
    )(xp, w, dinv_col)


def _tc_mid(parts, g1, dinv_col, b1row, w2):
    n = w2.shape[1]

    def body(p_ref, g_ref, d_ref, b_ref, w_ref, o_ref):
        d = d_ref[...]
        z = (p_ref[0] + p_ref[1] + g_ref[...]) * d + b_ref[...]
        z = jnp.maximum(z, 0.0)
        o_ref[...] = jnp.dot(z, w_ref[...], preferred_element_type=jnp.float32) * d

    return pl.pallas_call(
        body,
        grid=(NPAD // _BM,),
        in_specs=[
            pl.BlockSpec((2, _BM, D_HID), lambda i: (0, i, 0)),
            pl.BlockSpec((_BM, D_HID), lambda i: (i, 0)),
            pl.BlockSpec((_BM, 1), lambda i: (i, 0)),
            pl.BlockSpec((1, D_HID), lambda i: (0, 0)),
            pl.BlockSpec((D_HID, n), lambda i: (0, 0)),
        ],
        out_specs=pl.BlockSpec((_BM, n), lambda i: (i, 0)),
        out_shape=jax.ShapeDtypeStruct((NPAD, n), jnp.float32),
    )(parts, g1, dinv_col, b1row, w2)


def _tc_final(parts, g2, dinv_col, b2row):
    n = g2.shape[1]

    def body(p_ref, g_ref, d_ref, b_ref, o_ref):
        o_ref[...] = (p_ref[0] + p_ref[1] + g_ref[...]) * d_ref[...] + b_ref[...]

    return pl.pallas_call(
        body,
        grid=(NPAD // _BM,),
        in_specs=[
            pl.BlockSpec((2, _BM, n), lambda i: (0, i, 0)),
            pl.BlockSpec((_BM, n), lambda i: (i, 0)),
            pl.BlockSpec((_BM, 1), lambda i: (i, 0)),
            pl.BlockSpec((1, n), lambda i: (0, 0)),
        ],
        out_specs=pl.BlockSpec((_BM, n), lambda i: (i, 0)),
        out_shape=jax.ShapeDtypeStruct((NPAD, n), jnp.float32),
    )(parts, g2, dinv_col, b2row)


@jax.jit
def kernel(x, edge_index, W1, b1, W2, b2):
    ei = edge_index.astype(jnp.int32)
    src1d = ei[0]
    dst1d = ei[1]
    xp = jnp.pad(x, ((0, NPAD - N_NODES), (0, 0)))

    deg = _deg_kernel(dst1d)
    dinv_col = lax.rsqrt(deg[0] + deg[1] + 1.0)[:, None]

    g1 = _tc_scale_matmul(xp, W1, dinv_col)
    parts1 = _edge_scatter(g1, src1d, dst1d, D_HID)

    w2p = jnp.pad(W2, ((0, 0), (0, CW - N_CLASSES)))
    b2p = jnp.pad(b2, (0, CW - N_CLASSES))
    g2 = _tc_mid(parts1, g1, dinv_col, b1[None, :], w2p)
    parts2 = _edge_scatter(g2, src1d, dst1d, CW)

    outp = _tc_final(parts2, g2, dinv_col, b2p[None, :])
    return outp[:N_NODES, :N_CLASSES]


# speedup vs baseline: 28.8894x; 1.0022x over previous
import functools

import jax
import jax.numpy as jnp
from jax import lax
from jax.experimental import pallas as pl
from jax.experimental.pallas import tpu as pltpu
from jax.experimental.pallas import tpu_sc as plsc

N_NODES = 10000
N_EDGES = 320000
D_IN = 128
D_HID = 128
N_CLASSES = 40

NPAD = 10240
CW = 128
CHUNK = 128
NW = 32
NCHUNK = N_EDGES // CHUNK
CPT = NCHUNK // NW
REM = NCHUNK - CPT * NW
NSUB = 16
RPT = NPAD // NSUB


def _zero_rows(buf, nrows, width):
    def outer(r, _):
        def inner(j, _):
            buf[r, pl.ds(j * 16, 16)] = jnp.zeros((16,), jnp.float32)
            return 0
        lax.fori_loop(0, width // 16, inner, 0)
        return 0
    lax.fori_loop(0, nrows, outer, 0)


def _deg_kernel(dst1d):
    mesh = plsc.VectorSubcoreMesh(core_axis_name="c", subcore_axis_name="s")

    @functools.partial(
        pl.kernel,
        out_type=jax.ShapeDtypeStruct((2, NPAD), jnp.float32),
        mesh=mesh,
        scratch_types=[
            pltpu.VMEM((CPT, CHUNK), jnp.int32),
            pltpu.VMEM((CHUNK,), jnp.int32),
            pltpu.VMEM((CHUNK,), jnp.float32),
            pltpu.VMEM((RPT,), jnp.float32),
            pltpu.SemaphoreType.DMA,
            pltpu.SemaphoreType.DMA,
            pltpu.VMEM_SHARED((NPAD,), jnp.float32),
        ],
    )
    def deg(dst_hbm, out_hbm, di2, dx, ones_v, row_v, sem, sem2, acc):
        c = lax.axis_index("c")
        s = lax.axis_index("s")
        wid = c * NSUB + s
        e0 = wid * CPT * CHUNK

        def ld(t, _):
            pltpu.async_copy(
                dst_hbm.at[pl.ds(e0 + t * CHUNK, CHUNK)], di2.at[t], sem)
            return 0
        lax.fori_loop(0, CPT, ld, 0)

        def fill(i, _):
            ones_v[pl.ds(i * 16, 16)] = jnp.ones((16,), jnp.float32)
            return 0
        lax.fori_loop(0, CHUNK // 16, fill, 0)

        def fill0(i, _):
            row_v[pl.ds(i * 16, 16)] = jnp.zeros((16,), jnp.float32)
            return 0
        lax.fori_loop(0, RPT // 16, fill0, 0)
        pltpu.sync_copy(row_v, acc.at[pl.ds(s * RPT, RPT)])

        def dr(t, _):
            pltpu.make_async_copy(
                dst_hbm.at[pl.ds(e0, CHUNK)], di2.at[t], sem).wait()
            return 0
        lax.fori_loop(0, CPT, dr, 0)
        plsc.subcore_barrier()

        def sc(t, _):
            pltpu.async_copy(ones_v, acc.at[di2.at[t]], sem2, add=True)
            return 0
        lax.fori_loop(0, CPT, sc, 0)

        @pl.when(wid < REM)
        def _():
            base = (NW * CPT + wid) * CHUNK
            pltpu.sync_copy(dst_hbm.at[pl.ds(base, CHUNK)], dx)
            pltpu.sync_copy(ones_v, acc.at[dx], add=True)

        def dr2(t, _):
            pltpu.make_async_copy(ones_v, acc.at[di2.at[t]], sem2).wait()
            return 0
        lax.fori_loop(0, CPT, dr2, 0)
        plsc.subcore_barrier()
        pltpu.sync_copy(acc.at[pl.ds(s * RPT, RPT)], row_v)
        pltpu.sync_copy(row_v, out_hbm.at[c, pl.ds(s * RPT, RPT)])

    return deg(dst1d)


def _edge_scatter(g, src1d, dst1d, width):
    mesh = plsc.VectorSubcoreMesh(core_axis_name="c", subcore_axis_name="s")

    @functools.partial(
        pl.kernel,
        out_type=jax.ShapeDtypeStruct((2, NPAD, width), jnp.float32),
        mesh=mesh,
        scratch_types=[
            pltpu.VMEM((CPT * CHUNK,), jnp.int32),
            pltpu.VMEM((CHUNK,), jnp.int32),
            [pltpu.VMEM((CHUNK,), jnp.int32)] * 2,
            [pltpu.VMEM((CHUNK, width), jnp.float32)] * 2,
            [pltpu.SemaphoreType.DMA] * 2,
            [pltpu.SemaphoreType.DMA] * 2,
            pltpu.VMEM_SHARED((NPAD, width), jnp.float32),
        ],
    )
    def scat(g_hbm, src_hbm, dst_hbm, out_hbm,
             sb, sx, dis, rowss, sgs, sds, acc):
        c = lax.axis_index("c")
        s = lax.axis_index("s")
        wid = c * NSUB + s
        e0 = wid * CPT * CHUNK
        DEPTH = 2

        _zero_rows(rowss[0], CHUNK, width)
        def z(k, _):
            pltpu.sync_copy(rowss[0], acc.at[pl.ds(s * RPT + k * CHUNK, CHUNK)])
            return 0
        lax.fori_loop(0, RPT // CHUNK, z, 0)
        plsc.subcore_barrier()

        pltpu.sync_copy(src_hbm.at[pl.ds(e0, CPT * CHUNK)], sb)

        def src_at(t):
            return sb.at[pl.ds(t * CHUNK, CHUNK)]

        def start_gather(t, b):
            pltpu.async_copy(g_hbm.at[src_at(t)], rowss[b], sgs[b])

        def start_dst(t, b):
            pltpu.async_copy(
                dst_hbm.at[pl.ds(e0 + t * CHUNK, CHUNK)], dis[b], sds[b])

        for b in range(DEPTH):
            start_dst(b, b)
            start_gather(b, b)

        def slot(t, b):
            pltpu.make_async_copy(g_hbm.at[src_at(t)], rowss[b], sgs[b]).wait()
            pltpu.make_async_copy(
                dst_hbm.at[pl.ds(e0, CHUNK)], dis[b], sds[b]).wait()
            pltpu.sync_copy(rowss[b], acc.at[dis[b]], add=True)

            @pl.when(t + DEPTH < CPT)
            def _():
                start_dst(t + DEPTH, b)
                start_gather(t + DEPTH, b)

        def body(i, _):
            t0 = DEPTH * i
            for b in range(DEPTH):
                slot(t0 + b, b)
            return 0
        lax.fori_loop(0, CPT // DEPTH, body, 0)
        for b in range(CPT - DEPTH * (CPT // DEPTH)):
            slot(DEPTH * (CPT // DEPTH) + b, b)

        @pl.when(wid < REM)
        def _():
            base = (NW * CPT + wid) * CHUNK
            pltpu.sync_copy(src_hbm.at[pl.ds(base, CHUNK)], sx)
            pltpu.sync_copy(dst_hbm.at[pl.ds(base, CHUNK)], dis[0])
            pltpu.async_copy(g_hbm.at[sx], rowss[0], sgs[0]).wait()
            pltpu.sync_copy(rowss[0], acc.at[dis[0]], add=True)

        plsc.subcore_barrier()

        def rd(k, _):
            r0 = s * RPT + k * CHUNK
            pltpu.sync_copy(acc.at[pl.ds(r0, CHUNK)], rowss[0])
            pltpu.sync_copy(rowss[0], out_hbm.at[c, pl.ds(r0, CHUNK)])
            return 0
        lax.fori_loop(0, RPT // CHUNK, rd, 0)

    return scat(g, src1d, dst1d)


_BM = 256


def _tc_scale_matmul(xp, w, dinv_col):
    k = xp.shape[1]
    n = w.shape[1]

    def body(x_ref, w_ref, d_ref, o_ref):
        h = jnp.dot(x_ref[...], w_ref[...], preferred_element_type=jnp.float32)
        o_ref[...] = h * d_ref[...]

    return pl.pallas_call(
        body,
        grid=(NPAD // _BM,),
        in_specs=[
            pl.BlockSpec((_BM, k), lambda i: (i, 0)),
            pl.BlockSpec((k, n), lambda i: (0, 0)),
            pl.BlockSpec((_BM, 1), lambda i: (i, 0)),
        ],
        out_specs=pl.BlockSpec((_BM, n), lambda i: (i, 0)),
        out_shape=jax.ShapeDtypeStruct((NPAD, n), jnp.float32),
    )(xp, w, dinv_col)


def _tc_mid(parts, g1, dinv_col, b1row, w2):
    n = w2.shape[1]

    def body(p_ref, g_ref, d_ref, b_ref, w_ref, o_ref):
        d = d_ref[...]
        z = (p_ref[0] + p_ref[1] + g_ref[...]) * d + b_ref[...]
        z = jnp.maximum(z, 0.0)
        o_ref[...] = jnp.dot(z, w_ref[...], preferred_element_type=jnp.float32) * d

    return pl.pallas_call(
        body,
        grid=(NPAD // _BM,),
        in_specs=[
            pl.BlockSpec((2, _BM, D_HID), lambda i: (0, i, 0)),
            pl.BlockSpec((_BM, D_HID), lambda i: (i, 0)),
            pl.BlockSpec((_BM, 1), lambda i: (i, 0)),
            pl.BlockSpec((1, D_HID), lambda i: (0, 0)),
            pl.BlockSpec((D_HID, n), lambda i: (0, 0)),
        ],
        out_specs=pl.BlockSpec((_BM, n), lambda i: (i, 0)),
        out_shape=jax.ShapeDtypeStruct((NPAD, n), jnp.float32),
    )(parts, g1, dinv_col, b1row, w2)


def _tc_final(parts, g2, dinv_col, b2row):
    n = g2.shape[1]

    def body(p_ref, g_ref, d_ref, b_ref, o_ref):
        o_ref[...] = (p_ref[0] + p_ref[1] + g_ref[...]) * d_ref[...] + b_ref[...]

    return pl.pallas_call(
        body,
        grid=(NPAD // _BM,),
        in_specs=[
            pl.BlockSpec((2, _BM, n), lambda i: (0, i, 0)),
            pl.BlockSpec((_BM, n), lambda i: (i, 0)),
            pl.BlockSpec((_BM, 1), lambda i: (i, 0)),
            pl.BlockSpec((1, n), lambda i: (0, 0)),
        ],
        out_specs=pl.BlockSpec((_BM, n), lambda i: (i, 0)),
        out_shape=jax.ShapeDtypeStruct((NPAD, n), jnp.float32),
    )(parts, g2, dinv_col, b2row)


@jax.jit
def kernel(x, edge_index, W1, b1, W2, b2):
    ei = edge_index.astype(jnp.int32)
    src1d = ei[0]
    dst1d = ei[1]
    xp = jnp.pad(x, ((0, NPAD - N_NODES), (0, 0)))

    deg = _deg_kernel(dst1d)
    dinv_col = lax.rsqrt(deg[0] + deg[1] + 1.0)[:, None]

    g1 = _tc_scale_matmul(xp, W1, dinv_col)
    parts1 = _edge_scatter(g1, src1d, dst1d, D_HID)

    w2p = jnp.pad(W2, ((0, 0), (0, CW - N_CLASSES)))
    b2p = jnp.pad(b2, (0, CW - N_CLASSES))
    g2 = _tc_mid(parts1, g1, dinv_col, b1[None, :], w2p)
    parts2 = _edge_scatter(g2, src1d, dst1d, CW)

    outp = _tc_final(parts2, g2, dinv_col, b2p[None, :])
    return outp[:N_NODES, :N_CLASSES]
